# baseline (speedup 1.0000x reference)
import functools

import jax
import jax.numpy as jnp
from jax.experimental import pallas as pl
from jax.experimental.pallas import tpu as pltpu

_N = 10000
_D_IN = 128
_D_OUT = 128


def _seq_fts_body(seq_ref, wt_ref, out_ref, out_bf_ref):
    sf = jnp.dot(seq_ref[:, :], wt_ref[:, :], preferred_element_type=jnp.float32)
    out_ref[:, :] = sf
    out_bf_ref[:, :] = sf.astype(jnp.bfloat16)


def _main_body(adj_ref, sfb_ref, sf_ref, nt_ref, gtab_ref, btab_ref,
               bias_ref, a_ref, out_ref, *, bm):
    i = pl.program_id(0)
    acc = jnp.dot(adj_ref[:, :].astype(jnp.bfloat16), sfb_ref[:, :],
                  preferred_element_type=jnp.float32)
    t = nt_ref[:, :]
    gamma = jnp.where(t == 0.0, gtab_ref[0:1, :], gtab_ref[1:2, :])
    beta = jnp.where(t == 0.0, btab_ref[0:1, :], btab_ref[1:2, :])
    sf_blk = sf_ref[pl.ds(i * bm, bm), :]
    o = gamma * acc + beta + bias_ref[:, :] + sf_blk
    alpha = a_ref[0, 0]
    out_ref[:, :] = jnp.where(o >= 0.0, o, alpha * o)


def kernel(seq, adj, node_type, W, Wg, bg, Wb, bb, bias, a):
    n, d_in = seq.shape
    d_out = W.shape[0]

    wt = W.T
    gtab = Wg.T + bg[None, :]
    btab = Wb.T + bb[None, :]
    nt = node_type.astype(jnp.float32).reshape(n, 1)
    bias2 = bias.reshape(1, d_out)
    a2 = a.reshape(1, 1)

    bm1 = 2000
    sf, sf_bf = pl.pallas_call(
        _seq_fts_body,
        grid=(n // bm1,),
        in_specs=[
            pl.BlockSpec((bm1, d_in), lambda i: (i, 0)),
            pl.BlockSpec((d_in, d_out), lambda i: (0, 0)),
        ],
        out_specs=[
            pl.BlockSpec((bm1, d_out), lambda i: (i, 0)),
            pl.BlockSpec((bm1, d_out), lambda i: (i, 0)),
        ],
        out_shape=[
            jax.ShapeDtypeStruct((n, d_out), jnp.float32),
            jax.ShapeDtypeStruct((n, d_out), jnp.bfloat16),
        ],
    )(seq, wt)

    bm = 400
    out = pl.pallas_call(
        functools.partial(_main_body, bm=bm),
        grid=(n // bm,),
        in_specs=[
            pl.BlockSpec((bm, n), lambda i: (i, 0)),
            pl.BlockSpec((n, d_out), lambda i: (0, 0)),
            pl.BlockSpec((n, d_out), lambda i: (0, 0)),
            pl.BlockSpec((bm, 1), lambda i: (i, 0)),
            pl.BlockSpec((2, d_out), lambda i: (0, 0)),
            pl.BlockSpec((2, d_out), lambda i: (0, 0)),
            pl.BlockSpec((1, d_out), lambda i: (0, 0)),
            pl.BlockSpec((1, 1), lambda i: (0, 0)),
        ],
        out_specs=pl.BlockSpec((bm, d_out), lambda i: (i, 0)),
        out_shape=jax.ShapeDtypeStruct((n, d_out), jnp.float32),
        compiler_params=pltpu.CompilerParams(
            dimension_semantics=("arbitrary",),
        ),
    )(adj, sf_bf, sf, nt, gtab, btab, bias2, a2)
    return out
